# Initial kernel scaffold; baseline (speedup 1.0000x reference)
import jax
import jax.numpy as jnp
from jax.experimental import pallas as pl


def kernel(x, laplacian, weight, bias):
    raise NotImplementedError("write your pallas kernel here")


# speedup vs baseline: 1.0815x; 1.0815x over previous
import functools

import jax
import jax.numpy as jnp
from jax.experimental import pallas as pl
from jax.experimental.pallas import tpu as pltpu


def _pass1_kernel(l_ref, xb_ref, y_ref, yb_ref, lb_ref):
    lb = l_ref[...].astype(jnp.bfloat16)
    lb_ref[...] = lb
    y = jnp.dot(lb, xb_ref[...], preferred_element_type=jnp.float32)
    y_ref[...] = y
    yb_ref[...] = y.astype(jnp.bfloat16)


def _pass2_kernel(lb_ref, xb_ref, xprev_ref, y_ref, yb_ref):
    acc = jnp.dot(lb_ref[...], xb_ref[...], preferred_element_type=jnp.float32)
    y = 2.0 * acc - xprev_ref[...]
    y_ref[...] = y
    yb_ref[...] = y.astype(jnp.bfloat16)


def _pass3_kernel(lb_ref, x2b_ref, x0_ref, x1_ref, x2_ref, w_ref, b_ref,
                  out_ref, *, bm, f_out, pool):
    acc = jnp.dot(lb_ref[...], x2b_ref[...], preferred_element_type=jnp.float32)
    x3 = 2.0 * acc - x1_ref[...]
    t = jnp.dot(x0_ref[...], w_ref[0], preferred_element_type=jnp.float32)
    t = t + jnp.dot(x1_ref[...], w_ref[1], preferred_element_type=jnp.float32)
    t = t + jnp.dot(x2_ref[...], w_ref[2], preferred_element_type=jnp.float32)
    t = t + jnp.dot(x3, w_ref[3], preferred_element_type=jnp.float32)
    t = jnp.maximum(t + b_ref[...], 0.0)
    out_ref[...] = jnp.max(t.reshape(bm // pool, pool, f_out), axis=1)


_BM = 256


def kernel(x, laplacian, weight, bias):
    B, N, F_in = x.shape
    F_out = weight.shape[-1]
    degree = weight.shape[0] // F_in
    pool = 2
    bm = _BM
    nm = N // bm

    x0 = jnp.transpose(x, (1, 2, 0)).reshape(N, F_in * B)
    c = x0.shape[1]
    x0b = x0.astype(jnp.bfloat16)
    w4 = jnp.transpose(weight.reshape(F_in, degree, F_out), (1, 0, 2))
    b2 = bias.reshape(1, F_out)

    params = pltpu.CompilerParams(dimension_semantics=("arbitrary",))

    l_spec = pl.BlockSpec((bm, N), lambda i: (i, 0))
    vfull_spec = pl.BlockSpec((N, c), lambda i: (0, 0))
    vrow_spec = pl.BlockSpec((bm, c), lambda i: (i, 0))

    x1, x1b, lb = pl.pallas_call(
        _pass1_kernel,
        grid=(nm,),
        in_specs=[l_spec, vfull_spec],
        out_specs=[vrow_spec, vrow_spec, l_spec],
        out_shape=[
            jax.ShapeDtypeStruct((N, c), jnp.float32),
            jax.ShapeDtypeStruct((N, c), jnp.bfloat16),
            jax.ShapeDtypeStruct((N, N), jnp.bfloat16),
        ],
        compiler_params=params,
    )(laplacian, x0b)

    x2, x2b = pl.pallas_call(
        _pass2_kernel,
        grid=(nm,),
        in_specs=[l_spec, vfull_spec, vrow_spec],
        out_specs=[vrow_spec, vrow_spec],
        out_shape=[
            jax.ShapeDtypeStruct((N, c), jnp.float32),
            jax.ShapeDtypeStruct((N, c), jnp.bfloat16),
        ],
        compiler_params=params,
    )(lb, x1b, x0)

    out = pl.pallas_call(
        functools.partial(_pass3_kernel, bm=bm, f_out=F_out, pool=pool),
        grid=(nm,),
        in_specs=[
            l_spec,
            vfull_spec,
            vrow_spec,
            vrow_spec,
            vrow_spec,
            pl.BlockSpec((degree, F_in, F_out), lambda i: (0, 0, 0)),
            pl.BlockSpec((1, F_out), lambda i: (0, 0)),
        ],
        out_specs=pl.BlockSpec((bm // pool, F_out), lambda i: (i, 0)),
        out_shape=jax.ShapeDtypeStruct((N // pool, F_out), jnp.float32),
        compiler_params=params,
    )(lb, x2b, x0, x1, x2, w4, b2)

    return out.reshape(B, N // pool, F_out)


# speedup vs baseline: 1.1807x; 1.0918x over previous
import functools

import jax
import jax.numpy as jnp
from jax.experimental import pallas as pl
from jax.experimental.pallas import tpu as pltpu


def _pass1_kernel(l_ref, xb_ref, y_ref, yb_ref, lb_ref):
    lb = l_ref[...].astype(jnp.bfloat16)
    lb_ref[...] = lb
    y = jnp.dot(lb, xb_ref[...], preferred_element_type=jnp.float32)
    y_ref[...] = y
    yb_ref[...] = y.astype(jnp.bfloat16)


def _pass2_kernel(lb_ref, xb_ref, xprev_ref, y_ref, yb_ref):
    acc = jnp.dot(lb_ref[...], xb_ref[...], preferred_element_type=jnp.float32)
    y = 2.0 * acc - xprev_ref[...]
    y_ref[...] = y
    yb_ref[...] = y.astype(jnp.bfloat16)


def _pass3_kernel(lb_ref, x2b_ref, x0_ref, x1_ref, x2_ref, w_ref, b_ref,
                  out_ref, *, bm, f_out, pool):
    acc = jnp.dot(lb_ref[...], x2b_ref[...], preferred_element_type=jnp.float32)
    x3 = 2.0 * acc - x1_ref[...]
    t = jnp.dot(x0_ref[...], w_ref[0], preferred_element_type=jnp.float32)
    t = t + jnp.dot(x1_ref[...], w_ref[1], preferred_element_type=jnp.float32)
    t = t + jnp.dot(x2_ref[...], w_ref[2], preferred_element_type=jnp.float32)
    t = t + jnp.dot(x3, w_ref[3], preferred_element_type=jnp.float32)
    t = jnp.maximum(t + b_ref[...], 0.0)
    out_ref[...] = jnp.max(t.reshape(bm // pool, pool, f_out), axis=1)


_BM = 512


def kernel(x, laplacian, weight, bias):
    B, N, F_in = x.shape
    F_out = weight.shape[-1]
    degree = weight.shape[0] // F_in
    pool = 2
    bm = _BM
    nm = N // bm

    x0 = jnp.transpose(x, (1, 2, 0)).reshape(N, F_in * B)
    c = x0.shape[1]
    x0b = x0.astype(jnp.bfloat16)
    w4 = jnp.transpose(weight.reshape(F_in, degree, F_out), (1, 0, 2))
    b2 = bias.reshape(1, F_out)

    params = pltpu.CompilerParams(dimension_semantics=("arbitrary",))

    l_spec = pl.BlockSpec((bm, N), lambda i: (i, 0))
    vfull_spec = pl.BlockSpec((N, c), lambda i: (0, 0))
    vrow_spec = pl.BlockSpec((bm, c), lambda i: (i, 0))

    x1, x1b, lb = pl.pallas_call(
        _pass1_kernel,
        grid=(nm,),
        in_specs=[l_spec, vfull_spec],
        out_specs=[vrow_spec, vrow_spec, l_spec],
        out_shape=[
            jax.ShapeDtypeStruct((N, c), jnp.float32),
            jax.ShapeDtypeStruct((N, c), jnp.bfloat16),
            jax.ShapeDtypeStruct((N, N), jnp.bfloat16),
        ],
        compiler_params=params,
    )(laplacian, x0b)

    x2, x2b = pl.pallas_call(
        _pass2_kernel,
        grid=(nm,),
        in_specs=[l_spec, vfull_spec, vrow_spec],
        out_specs=[vrow_spec, vrow_spec],
        out_shape=[
            jax.ShapeDtypeStruct((N, c), jnp.float32),
            jax.ShapeDtypeStruct((N, c), jnp.bfloat16),
        ],
        compiler_params=params,
    )(lb, x1b, x0)

    out = pl.pallas_call(
        functools.partial(_pass3_kernel, bm=bm, f_out=F_out, pool=pool),
        grid=(nm,),
        in_specs=[
            l_spec,
            vfull_spec,
            vrow_spec,
            vrow_spec,
            vrow_spec,
            pl.BlockSpec((degree, F_in, F_out), lambda i: (0, 0, 0)),
            pl.BlockSpec((1, F_out), lambda i: (0, 0)),
        ],
        out_specs=pl.BlockSpec((bm // pool, F_out), lambda i: (i, 0)),
        out_shape=jax.ShapeDtypeStruct((N // pool, F_out), jnp.float32),
        compiler_params=params,
    )(lb, x2b, x0, x1, x2, w4, b2)

    return out.reshape(B, N // pool, F_out)
